# baseline (speedup 1.0000x reference)
import functools

import jax
import jax.numpy as jnp
from jax import lax
from jax.experimental import pallas as pl
from jax.experimental.pallas import tpu as pltpu
from jax.experimental.pallas import tpu_sc as plsc

_B = 16384
_D = 64
_DP1 = 65
_MARGIN = 0.01

_NW = 32
_RW = _B // _NW
_CH = 128
_NCHUNK = _RW // _CH
_NG = _CH // 16
_NIDX = 13


def _sqrt16(x):
    xh = x * 0.5
    i = lax.bitcast_convert_type(x, jnp.int32)
    i = jnp.int32(0x5F3759DF) - (i >> 1)
    y = lax.bitcast_convert_type(i, jnp.float32)
    for _ in range(3):
        y = y * (1.5 - xh * y * y)
    return x * y


def _relu(x):
    return jnp.maximum(x, 0.0)


_mesh = plsc.VectorSubcoreMesh(core_axis_name="c", subcore_axis_name="s")


@functools.partial(
    pl.kernel,
    mesh=_mesh,
    compiler_params=pltpu.CompilerParams(
        needs_layout_passes=False, use_tc_tiling_on_sc=False),
    out_type=jax.ShapeDtypeStruct((_B,), jnp.float32),
    scratch_types=[
        pltpu.VMEM((_NIDX, _CH), jnp.int32),
        *[pltpu.VMEM((_CH, _DP1), jnp.float32) for _ in range(11)],
        *[pltpu.VMEM((_CH, _D), jnp.float32) for _ in range(2)],
        pltpu.VMEM((_CH,), jnp.float32),
        pltpu.SemaphoreType.DMA,
    ],
)
def _sc_kernel(idx_hbm, cls_hbm, rel_hbm, out_hbm,
               idx_v, c1a, c1b, c2a, c2b, c2c, c3c, c3d, c4c, c4d,
               cda, cdb, r3, r4, out_v, sem):
    wid = lax.axis_index("s") * 2 + lax.axis_index("c")
    cls_bufs = [c1a, c1b, c2a, c2b, c2c, c3c, c3d, c4c, c4d, cda, cdb]
    rel_bufs = [r3, r4]

    for ci in range(_NCHUNK):
        base = wid * _RW + ci * _CH
        pltpu.sync_copy(idx_hbm.at[:, pl.ds(base, _CH)], idx_v)
        handles = []
        for j in range(11):
            handles.append(
                pltpu.async_copy(cls_hbm.at[idx_v.at[j]], cls_bufs[j], sem))
        for j in range(2):
            handles.append(
                pltpu.async_copy(rel_hbm.at[idx_v.at[11 + j]], rel_bufs[j], sem))
        for h in handles:
            h.wait()

        def group_body(g, _):
            rows = lax.iota(jnp.int32, 16) + g * 16
            r65 = rows * _DP1
            r64 = rows * _D
            zeros = jnp.zeros((16,), jnp.float32)

            def gat(buf, off):
                colv = jnp.full((16,), 0, dtype=jnp.int32) + off
                return plsc.load_gather(buf, [rows, colv])

            def dloop(body, n_acc):
                def outer(j, accs):
                    for k in range(8):
                        d = j * 8 + k
                        accs = body(d, accs)
                    return accs
                return lax.fori_loop(0, _D // 8, outer, (zeros,) * n_acc)

            def b1(colv, accs):
                a_cd, a_c, a_d = accs
                cv = gat(c1a, colv)
                dv = gat(c1b, colv)
                df = cv - dv
                return (a_cd + df * df, a_c + cv * cv, a_d + dv * dv)
            a_cd, a_c, a_d = dloop(b1, 3)
            rc = jnp.abs(gat(c1a, _D))
            rd = jnp.abs(gat(c1b, _D))
            l1 = (_relu(_sqrt16(a_cd) + rc - rd - _MARGIN)
                  + jnp.abs(_sqrt16(a_c) - 1.0) + jnp.abs(_sqrt16(a_d) - 1.0))

            def b2(colv, accs):
                a1, a2, a3, ac, ad, ae = accs
                cv = gat(c2a, colv)
                dv = gat(c2b, colv)
                ev = gat(c2c, colv)
                d1 = dv - cv
                d2 = ev - cv
                d3 = ev - dv
                return (a1 + d1 * d1, a2 + d2 * d2, a3 + d3 * d3,
                        ac + cv * cv, ad + dv * dv, ae + ev * ev)
            a1, a2, a3, ac, ad, ae = dloop(b2, 6)
            rc = jnp.abs(gat(c2a, _D))
            rd = jnp.abs(gat(c2b, _D))
            l2 = (_relu(_sqrt16(a1) - (rc + rd))
                  + _relu(_sqrt16(a2) - rc) + _relu(_sqrt16(a3) - rd)
                  + jnp.abs(_sqrt16(ac) - 1.0) + jnp.abs(_sqrt16(ad) - 1.0)
                  + jnp.abs(_sqrt16(ae) - 1.0))

            def b3(colv, accs):
                a_e, a_cr, a_d3 = accs
                cv = gat(c3c, colv) + gat(r3, colv)
                dv = gat(c3d, colv)
                df = cv - dv
                return (a_e + df * df, a_cr + cv * cv, a_d3 + dv * dv)
            a_e, a_cr, a_d3 = dloop(b3, 3)
            rc = jnp.abs(gat(c3c, _D))
            rd = jnp.abs(gat(c3d, _D))
            l3 = (_relu(_sqrt16(a_e) + rc - rd - _MARGIN)
                  + jnp.abs(_sqrt16(a_cr) - 1.0) + jnp.abs(_sqrt16(a_d3) - 1.0))

            def b4(colv, accs):
                a_x, a_x1, a_d4 = accs
                xv = gat(c4c, colv) - gat(r4, colv)
                dv = gat(c4d, colv)
                df = dv - xv
                return (a_x + df * df, a_x1 + xv * xv, a_d4 + dv * dv)
            a_x, a_x1, a_d4 = dloop(b4, 3)
            rc = jnp.abs(gat(c4c, _D))
            rd = jnp.abs(gat(c4d, _D))
            l4 = (_relu(_sqrt16(a_x) - rc - rd - _MARGIN)
                  + jnp.abs(_sqrt16(a_x1) - 1.0) + jnp.abs(_sqrt16(a_d4) - 1.0))

            def b5(colv, accs):
                a_cd5, a_c5, a_d5 = accs
                cv = gat(cda, colv)
                dv = gat(cdb, colv)
                df = dv - cv
                return (a_cd5 + df * df, a_c5 + cv * cv, a_d5 + dv * dv)
            a_cd5, a_c5, a_d5 = dloop(b5, 3)
            rc = jnp.abs(gat(cda, _D))
            rd = jnp.abs(gat(cdb, _D))
            l5 = (_relu(rc + rd - _sqrt16(a_cd5) + _MARGIN)
                  + jnp.abs(_sqrt16(a_c5) - 1.0) + jnp.abs(_sqrt16(a_d5) - 1.0))

            out_v[pl.ds(g * 16, 16)] = l1 + l2 + l3 + l4 + l5
            return 0

        lax.fori_loop(0, _NG, group_body, 0)
        pltpu.sync_copy(out_v, out_hbm.at[pl.ds(base, _CH)])


def kernel(nf1, nf2, nf3, nf4, dis, neg, cls_emb, rel_emb):
    idx_all = jnp.stack([
        nf1[:, 0], nf1[:, 1],
        nf2[:, 0], nf2[:, 1], nf2[:, 2],
        nf3[:, 0], nf3[:, 2],
        nf4[:, 1], nf4[:, 2],
        dis[:, 0], dis[:, 1],
        nf3[:, 1], nf4[:, 0],
    ])
    out = _sc_kernel(idx_all, cls_emb, rel_emb)
    return out.reshape(_B, 1)
